# baseline (speedup 1.0000x reference)
import jax
import jax.numpy as jnp
from jax import lax
from jax.experimental import pallas as pl
from jax.experimental.pallas import tpu as pltpu

_EPS = 1e-5
_VMEM_LIMIT = 64 * 1024 * 1024


def _cparams():
    return pltpu.CompilerParams(
        dimension_semantics=("parallel", "parallel"),
        vmem_limit_bytes=_VMEM_LIMIT)


def _chunk_depth(d):
    assert d % 2 == 0
    return 4 if d % 4 == 0 else 2


def _split_cat(v):
    hi = v.astype(jnp.bfloat16)
    lo = (v - hi.astype(jnp.float32)).astype(jnp.bfloat16)
    return jnp.concatenate([hi, lo], axis=-1)


def _conv_kernel(xh_ref, xl_ref, xp_ref, xn_ref, w_ref, b_ref, sc_ref, sh_ref,
                 yh_ref, yl_ref, yp_ref, yn_ref, st_ref):
    td, h, w, cout = (yh_ref.shape[1], yh_ref.shape[2], yh_ref.shape[3],
                      yh_ref.shape[4])
    cin = xh_ref.shape[4]
    j = pl.program_id(1)
    nb = pl.num_programs(1)
    f32 = jnp.float32
    bf16 = jnp.bfloat16
    c2 = 2 * cin

    sc3 = sc_ref[...].reshape(1, 1, cin)
    sh3 = sh_ref[...].reshape(1, 1, cin)
    sc4 = sc_ref[...].reshape(1, 1, 1, cin)
    sh4 = sh_ref[...].reshape(1, 1, 1, cin)

    prev = jnp.where(j > 0, xp_ref[0, 0], 0.0)
    nxt = jnp.where(j < nb - 1, xn_ref[0, 0], 0.0)
    prev = _split_cat((prev * sc3 + sh3) * (j > 0).astype(f32))[None]
    nxt = _split_cat((nxt * sc3 + sh3) * (j < nb - 1).astype(f32))[None]
    xc = xh_ref[0].astype(f32) + xl_ref[0].astype(f32)
    ctr = _split_cat(xc * sc4 + sh4)

    xd = jnp.concatenate([prev, ctr, nxt], axis=0)
    zrow = jnp.zeros((td + 2, 1, w, c2), bf16)
    xh = jnp.concatenate([zrow, xd, zrow], axis=1)
    zcol = jnp.zeros((td + 2, h + 2, 1, c2), bf16)
    xpw = jnp.concatenate([zcol, xh, zcol], axis=2)
    xw = jnp.concatenate([xpw[:, :, 0:w, :],
                          xpw[:, :, 1:w + 1, :],
                          xpw[:, :, 2:w + 2, :]], axis=3)

    acc = jnp.zeros((td * h * w, cout), dtype=f32)
    for kd in range(3):
        for kh in range(3):
            a = xw[kd:kd + td, kh:kh + h].reshape(td * h * w, 3 * c2)
            acc = acc + jnp.dot(a, w_ref[kd * 3 + kh],
                                preferred_element_type=f32)
    acc = jnp.maximum(acc + b_ref[...], 0.0)
    yv = acc.reshape(td, h, w, cout)
    yhi = yv.astype(bf16)
    yh_ref[...] = yhi[None]
    yl_ref[...] = (yv - yhi.astype(f32)).astype(bf16)[None]
    yp_ref[...] = yv[td - 1:td][None]
    yn_ref[...] = yv[0:1][None]
    st_ref[0, 0, 0:1, :] = jnp.sum(acc, axis=0, keepdims=True)
    st_ref[0, 0, 1:2, :] = jnp.sum(acc * acc, axis=0, keepdims=True)


def _bn_pool_kernel(y_ref, yl_ref, sc_ref, sh_ref, f_ref, p_ref, pp_ref,
                    pn_ref, scr_ref):
    td, h, w, c = y_ref.shape[1], y_ref.shape[2], y_ref.shape[3], y_ref.shape[4]
    t2, h2, w2 = td // 2, h // 2, w // 2
    sc = sc_ref[...].reshape(1, 1, 1, c)
    sh = sh_ref[...].reshape(1, 1, 1, c)
    yv = y_ref[0].astype(jnp.float32) + yl_ref[0].astype(jnp.float32)
    yn = yv * sc + sh
    f_ref[...] = jnp.transpose(yn.reshape(td * h * w, c), (1, 0)
                               ).reshape(1, c, td * h * w)
    a = yn.reshape(t2, 2, h, w, c)
    a = jnp.maximum(a[:, 0], a[:, 1])
    b = a.reshape(t2, h2, 2, w, c)
    b = jnp.maximum(b[:, :, 0], b[:, :, 1])
    scr_ref[...] = b.reshape(t2 * h2, w, c)
    pooled = jnp.maximum(scr_ref[:, pl.ds(0, w2, 2), :],
                         scr_ref[:, pl.ds(1, w2, 2), :])
    pv = pooled.reshape(t2, h2, w2, c)
    p_ref[...] = pv[None]
    pp_ref[...] = pv[t2 - 1:t2][None]
    pn_ref[...] = pv[0:1][None]


def _bn_only_kernel(y_ref, yl_ref, sc_ref, sh_ref, f_ref):
    td, h, w, c = y_ref.shape[1], y_ref.shape[2], y_ref.shape[3], y_ref.shape[4]
    sc = sc_ref[...].reshape(1, 1, 1, c)
    sh = sh_ref[...].reshape(1, 1, 1, c)
    yv = y_ref[0].astype(jnp.float32) + yl_ref[0].astype(jnp.float32)
    yn = yv * sc + sh
    f_ref[...] = jnp.transpose(yn.reshape(td * h * w, c), (1, 0)
                               ).reshape(1, c, td * h * w)


def _conv3d_relu_stats(xh, xl, xp, xn, wt, bias, in_scale, in_shift, td, r=1):
    n, d, h, w, cin = xh.shape
    cout = wt.shape[-1]
    nb = d // td
    yh, yl, yp, yn, st = pl.pallas_call(
        _conv_kernel,
        out_shape=(jax.ShapeDtypeStruct((n, d, h, w, cout), jnp.bfloat16),
                   jax.ShapeDtypeStruct((n, d, h, w, cout), jnp.bfloat16),
                   jax.ShapeDtypeStruct((n, nb + 1, h, w, cout), jnp.float32),
                   jax.ShapeDtypeStruct((n, nb + 1, h, w, cout), jnp.float32),
                   jax.ShapeDtypeStruct((n, nb, 2, cout), jnp.float32)),
        grid=(n, nb),
        in_specs=[
            pl.BlockSpec((1, td, h, w, cin), lambda i, j: (i, j, 0, 0, 0)),
            pl.BlockSpec((1, td, h, w, cin), lambda i, j: (i, j, 0, 0, 0)),
            pl.BlockSpec((1, 1, h, w, cin), lambda i, j: (i, r * j, 0, 0, 0)),
            pl.BlockSpec((1, 1, h, w, cin),
                         lambda i, j: (i, r * j + r, 0, 0, 0)),
            pl.BlockSpec((9, 6 * cin, cout), lambda i, j: (0, 0, 0)),
            pl.BlockSpec((1, cout), lambda i, j: (0, 0)),
            pl.BlockSpec((1, cin), lambda i, j: (0, 0)),
            pl.BlockSpec((1, cin), lambda i, j: (0, 0)),
        ],
        out_specs=(
            pl.BlockSpec((1, td, h, w, cout), lambda i, j: (i, j, 0, 0, 0)),
            pl.BlockSpec((1, td, h, w, cout), lambda i, j: (i, j, 0, 0, 0)),
            pl.BlockSpec((1, 1, h, w, cout), lambda i, j: (i, j + 1, 0, 0, 0)),
            pl.BlockSpec((1, 1, h, w, cout), lambda i, j: (i, j, 0, 0, 0)),
            pl.BlockSpec((1, 1, 2, cout), lambda i, j: (i, j, 0, 0)),
        ),
        compiler_params=_cparams(),
    )(xh, xl, xp, xn, wt, bias, in_scale, in_shift)
    return yh, yl, yp, yn, st


def _bn_pool(yh, yl, scale, shift, td):
    n, d, h, w, c = yh.shape
    nb = d // td
    t2 = td // 2
    feat, pooled, pp, pn = pl.pallas_call(
        _bn_pool_kernel,
        out_shape=(jax.ShapeDtypeStruct((n, c, d * h * w), jnp.float32),
                   jax.ShapeDtypeStruct((n, d // 2, h // 2, w // 2, c),
                                        jnp.float32),
                   jax.ShapeDtypeStruct((n, nb + 1, h // 2, w // 2, c),
                                        jnp.float32),
                   jax.ShapeDtypeStruct((n, nb + 1, h // 2, w // 2, c),
                                        jnp.float32)),
        grid=(n, nb),
        in_specs=[
            pl.BlockSpec((1, td, h, w, c), lambda i, j: (i, j, 0, 0, 0)),
            pl.BlockSpec((1, td, h, w, c), lambda i, j: (i, j, 0, 0, 0)),
            pl.BlockSpec((1, c), lambda i, j: (0, 0)),
            pl.BlockSpec((1, c), lambda i, j: (0, 0)),
        ],
        out_specs=(
            pl.BlockSpec((1, c, td * h * w), lambda i, j: (i, 0, j)),
            pl.BlockSpec((1, t2, h // 2, w // 2, c),
                         lambda i, j: (i, j, 0, 0, 0)),
            pl.BlockSpec((1, 1, h // 2, w // 2, c),
                         lambda i, j: (i, j + 1, 0, 0, 0)),
            pl.BlockSpec((1, 1, h // 2, w // 2, c),
                         lambda i, j: (i, j, 0, 0, 0)),
        ),
        scratch_shapes=[pltpu.VMEM(((td // 2) * (h // 2), w, c), jnp.float32)],
        compiler_params=_cparams(),
    )(yh, yl, scale, shift)
    return feat.reshape(n, c, d, h, w), pooled, pp, pn


def _bn_only(yh, yl, scale, shift, td):
    n, d, h, w, c = yh.shape
    nb = d // td
    feat = pl.pallas_call(
        _bn_only_kernel,
        out_shape=jax.ShapeDtypeStruct((n, c, d * h * w), jnp.float32),
        grid=(n, nb),
        in_specs=[
            pl.BlockSpec((1, td, h, w, c), lambda i, j: (i, j, 0, 0, 0)),
            pl.BlockSpec((1, td, h, w, c), lambda i, j: (i, j, 0, 0, 0)),
            pl.BlockSpec((1, c), lambda i, j: (0, 0)),
            pl.BlockSpec((1, c), lambda i, j: (0, 0)),
        ],
        out_specs=pl.BlockSpec((1, c, td * h * w), lambda i, j: (i, 0, j)),
        compiler_params=_cparams(),
    )(yh, yl, scale, shift)
    return feat.reshape(n, c, d, h, w)


def _bn_affine(st, gamma, beta, count):
    s = jnp.sum(st, axis=(0, 1))
    mean = s[0] / count
    var = jnp.maximum(s[1] / count - mean * mean, 0.0)
    scale = gamma * lax.rsqrt(var + _EPS)
    shift = beta - mean * scale
    return scale.reshape(1, -1), shift.reshape(1, -1)


def _pack_w(wgt):
    co, ci = wgt.shape[0], wgt.shape[1]
    wb = jnp.transpose(wgt, (2, 3, 4, 1, 0)).astype(jnp.bfloat16)
    w2 = jnp.concatenate([wb, wb], axis=3)
    return w2.reshape(9, 6 * ci, co)


def _halo_slabs(x, td):
    n, d, h, w, c = x.shape
    zp = jnp.zeros((n, 1, h, w, c), x.dtype)
    xp = jnp.concatenate([zp, x[:, td - 1::td]], axis=1)
    xn = jnp.concatenate([x[:, ::td], zp], axis=1)
    return xp, xn


def _block(xh, xl, xp, xn, r, w1, b1, w2, b2, gamma, beta, last):
    n, d, h, w, cin = xh.shape
    td = _chunk_depth(d)
    cout = w1.shape[0]
    count = float(n * d * h * w)
    one = jnp.ones((1, cin), jnp.float32)
    zero = jnp.zeros((1, cin), jnp.float32)
    b1r = b1.reshape(1, cout).astype(jnp.float32)
    b2r = b2.reshape(1, cout).astype(jnp.float32)

    y1h, y1l, y1p, y1n, st1 = _conv3d_relu_stats(xh, xl, xp, xn, _pack_w(w1),
                                                 b1r, one, zero, td, r)
    sc1, sh1 = _bn_affine(st1, gamma, beta, count)
    y2h, y2l, _, _, st2 = _conv3d_relu_stats(y1h, y1l, y1p, y1n, _pack_w(w2),
                                             b2r, sc1, sh1, td, 1)
    sc2, sh2 = _bn_affine(st2, gamma, beta, count)
    if last:
        return _bn_only(y2h, y2l, sc2, sh2, td), None, None, None
    feat, pooled, pp, pn = _bn_pool(y2h, y2l, sc2, sh2, td)
    return feat, pooled, pp, pn


def kernel(x, b0_w1, b0_b1, b0_w2, b0_b2, b0_gamma, b0_beta,
           b1_w1, b1_b1, b1_w2, b1_b2, b1_gamma, b1_beta):
    xc = jnp.transpose(x, (0, 2, 3, 4, 1))
    td0 = _chunk_depth(xc.shape[1])
    xp0, xn0 = _halo_slabs(xc, td0)
    xch = xc.astype(jnp.bfloat16)
    xcl = (xc - xch.astype(jnp.float32)).astype(jnp.bfloat16)
    f0, pooled, pp, pn = _block(xch, xcl, xp0, xn0, 1, b0_w1, b0_b1, b0_w2,
                                b0_b2, b0_gamma, b0_beta, last=False)
    ph = pooled.astype(jnp.bfloat16)
    plo = (pooled - ph.astype(jnp.float32)).astype(jnp.bfloat16)
    r1 = _chunk_depth(pooled.shape[1]) // (td0 // 2)
    f1, _, _, _ = _block(ph, plo, pp, pn, r1, b1_w1, b1_b1, b1_w2, b1_b2,
                         b1_gamma, b1_beta, last=True)
    return [f0, f1]


# speedup vs baseline: 1.2315x; 1.2315x over previous
import jax
import jax.numpy as jnp
from jax import lax
from jax.experimental import pallas as pl
from jax.experimental.pallas import tpu as pltpu

_EPS = 1e-5
_VMEM_LIMIT = 64 * 1024 * 1024


def _cparams():
    return pltpu.CompilerParams(
        dimension_semantics=("parallel", "parallel"),
        vmem_limit_bytes=_VMEM_LIMIT)


def _chunk_depth(d):
    assert d % 2 == 0
    return 4 if d % 4 == 0 else 2


def _split_hl(v):
    hi = v.astype(jnp.bfloat16)
    lo = (v - hi.astype(jnp.float32)).astype(jnp.bfloat16)
    return hi, lo


def _conv_kernel(xp_ref, xc_ref, xn_ref, w_ref, b_ref, sc_ref, sh_ref,
                 y_ref, st_ref):
    td, h, w, cout = y_ref.shape[1], y_ref.shape[2], y_ref.shape[3], y_ref.shape[4]
    cin = xc_ref.shape[4]
    j = pl.program_id(1)
    nb = pl.num_programs(1)
    f32 = jnp.float32
    bf16 = jnp.bfloat16

    sc3 = sc_ref[...].reshape(1, 1, cin)
    sh3 = sh_ref[...].reshape(1, 1, cin)
    sc4 = sc_ref[...].reshape(1, 1, 1, cin)
    sh4 = sh_ref[...].reshape(1, 1, 1, cin)

    prev = (xp_ref[0, 0] * sc3 + sh3) * (j > 0).astype(f32)
    nxt = (xn_ref[0, 0] * sc3 + sh3) * (j < nb - 1).astype(f32)
    ctr = xc_ref[0] * sc4 + sh4

    ph, plo = _split_hl(prev)
    nh, nlo = _split_hl(nxt)
    ch, clo = _split_hl(ctr)
    p2 = jnp.concatenate([ph, plo], axis=-1)[None]
    n2 = jnp.concatenate([nh, nlo], axis=-1)[None]
    c2 = jnp.concatenate([ch, clo], axis=-1)

    c2w = 2 * cin
    xd = jnp.concatenate([p2, c2, n2], axis=0)
    zrow = jnp.zeros((td + 2, 1, w, c2w), bf16)
    xh = jnp.concatenate([zrow, xd, zrow], axis=1)
    zcol = jnp.zeros((td + 2, h + 2, 1, c2w), bf16)
    xpw = jnp.concatenate([zcol, xh, zcol], axis=2)
    xw = jnp.concatenate([xpw[:, :, 0:w, :],
                          xpw[:, :, 1:w + 1, :],
                          xpw[:, :, 2:w + 2, :]], axis=3)

    acc = jnp.zeros((td * h * w, cout), dtype=f32)
    for kd in range(3):
        for kh in range(3):
            a = xw[kd:kd + td, kh:kh + h].reshape(td * h * w, 3 * c2w)
            acc = acc + jnp.dot(a, w_ref[kd * 3 + kh],
                                preferred_element_type=f32)
    acc = jnp.maximum(acc + b_ref[...], 0.0)
    y_ref[...] = acc.reshape(1, td, h, w, cout)
    st_ref[0, 0, 0:1, :] = jnp.sum(acc, axis=0, keepdims=True)
    st_ref[0, 0, 1:2, :] = jnp.sum(acc * acc, axis=0, keepdims=True)


def _bn_pool_kernel(y_ref, sc_ref, sh_ref, f_ref, p_ref, scr_ref):
    td, h, w, c = y_ref.shape[1], y_ref.shape[2], y_ref.shape[3], y_ref.shape[4]
    t2, h2, w2 = td // 2, h // 2, w // 2
    sc = sc_ref[...].reshape(1, 1, 1, c)
    sh = sh_ref[...].reshape(1, 1, 1, c)
    yn = y_ref[0] * sc + sh
    f_ref[...] = yn.reshape(1, td, h, w, c)
    a = yn.reshape(t2, 2, h, w, c)
    a = jnp.maximum(a[:, 0], a[:, 1])
    b = a.reshape(t2, h2, 2, w, c)
    b = jnp.maximum(b[:, :, 0], b[:, :, 1])
    scr_ref[...] = b.reshape(t2 * h2, w, c)
    pooled = jnp.maximum(scr_ref[:, pl.ds(0, w2, 2), :],
                         scr_ref[:, pl.ds(1, w2, 2), :])
    p_ref[...] = pooled.reshape(1, t2, h2, w2, c)


def _bn_only_kernel(y_ref, sc_ref, sh_ref, f_ref):
    td, h, w, c = y_ref.shape[1], y_ref.shape[2], y_ref.shape[3], y_ref.shape[4]
    sc = sc_ref[...].reshape(1, 1, 1, c)
    sh = sh_ref[...].reshape(1, 1, 1, c)
    f_ref[...] = (y_ref[0] * sc + sh).reshape(1, td, h, w, c)


def _conv3d_relu_stats(x, wt, bias, in_scale, in_shift, td):
    n, d, h, w, cin = x.shape
    cout = wt.shape[-1]
    nb = d // td

    def prev_map(i, j):
        return (i, jnp.maximum(j * td - 1, 0), 0, 0, 0)

    def next_map(i, j):
        return (i, jnp.minimum(j * td + td, d - 1), 0, 0, 0)

    y, st = pl.pallas_call(
        _conv_kernel,
        out_shape=(jax.ShapeDtypeStruct((n, d, h, w, cout), jnp.float32),
                   jax.ShapeDtypeStruct((n, nb, 2, cout), jnp.float32)),
        grid=(n, nb),
        in_specs=[
            pl.BlockSpec((1, 1, h, w, cin), prev_map),
            pl.BlockSpec((1, td, h, w, cin), lambda i, j: (i, j, 0, 0, 0)),
            pl.BlockSpec((1, 1, h, w, cin), next_map),
            pl.BlockSpec((9, 6 * cin, cout), lambda i, j: (0, 0, 0)),
            pl.BlockSpec((1, cout), lambda i, j: (0, 0)),
            pl.BlockSpec((1, cin), lambda i, j: (0, 0)),
            pl.BlockSpec((1, cin), lambda i, j: (0, 0)),
        ],
        out_specs=(
            pl.BlockSpec((1, td, h, w, cout), lambda i, j: (i, j, 0, 0, 0)),
            pl.BlockSpec((1, 1, 2, cout), lambda i, j: (i, j, 0, 0)),
        ),
        compiler_params=_cparams(),
    )(x, x, x, wt, bias, in_scale, in_shift)
    return y, st


def _bn_pool(y, scale, shift, td):
    n, d, h, w, c = y.shape
    nb = d // td
    feat, pooled = pl.pallas_call(
        _bn_pool_kernel,
        out_shape=(jax.ShapeDtypeStruct((n, d, h, w, c), jnp.float32),
                   jax.ShapeDtypeStruct((n, d // 2, h // 2, w // 2, c),
                                        jnp.float32)),
        grid=(n, nb),
        in_specs=[
            pl.BlockSpec((1, td, h, w, c), lambda i, j: (i, j, 0, 0, 0)),
            pl.BlockSpec((1, c), lambda i, j: (0, 0)),
            pl.BlockSpec((1, c), lambda i, j: (0, 0)),
        ],
        out_specs=(
            pl.BlockSpec((1, td, h, w, c), lambda i, j: (i, j, 0, 0, 0)),
            pl.BlockSpec((1, td // 2, h // 2, w // 2, c),
                         lambda i, j: (i, j, 0, 0, 0)),
        ),
        scratch_shapes=[pltpu.VMEM(((td // 2) * (h // 2), w, c), jnp.float32)],
        compiler_params=_cparams(),
    )(y, scale, shift)
    return feat, pooled


def _bn_only(y, scale, shift, td):
    n, d, h, w, c = y.shape
    nb = d // td
    return pl.pallas_call(
        _bn_only_kernel,
        out_shape=jax.ShapeDtypeStruct((n, d, h, w, c), jnp.float32),
        grid=(n, nb),
        in_specs=[
            pl.BlockSpec((1, td, h, w, c), lambda i, j: (i, j, 0, 0, 0)),
            pl.BlockSpec((1, c), lambda i, j: (0, 0)),
            pl.BlockSpec((1, c), lambda i, j: (0, 0)),
        ],
        out_specs=pl.BlockSpec((1, td, h, w, c), lambda i, j: (i, j, 0, 0, 0)),
        compiler_params=_cparams(),
    )(y, scale, shift)


def _bn_affine(st, gamma, beta, count):
    s = jnp.sum(st, axis=(0, 1))
    mean = s[0] / count
    var = jnp.maximum(s[1] / count - mean * mean, 0.0)
    scale = gamma * lax.rsqrt(var + _EPS)
    shift = beta - mean * scale
    return scale.reshape(1, -1), shift.reshape(1, -1)


def _pack_w(wgt):
    co, ci = wgt.shape[0], wgt.shape[1]
    wb = jnp.transpose(wgt, (2, 3, 4, 1, 0)).astype(jnp.bfloat16)
    w2 = jnp.concatenate([wb, wb], axis=3)
    return w2.reshape(9, 6 * ci, co)


def _block(x, w1, b1, w2, b2, gamma, beta, last):
    n, d, h, w, cin = x.shape
    td = _chunk_depth(d)
    cout = w1.shape[0]
    count = float(n * d * h * w)
    one = jnp.ones((1, cin), jnp.float32)
    zero = jnp.zeros((1, cin), jnp.float32)
    b1r = b1.reshape(1, cout).astype(jnp.float32)
    b2r = b2.reshape(1, cout).astype(jnp.float32)

    y1, st1 = _conv3d_relu_stats(x, _pack_w(w1), b1r, one, zero, td)
    sc1, sh1 = _bn_affine(st1, gamma, beta, count)
    y2, st2 = _conv3d_relu_stats(y1, _pack_w(w2), b2r, sc1, sh1, td)
    sc2, sh2 = _bn_affine(st2, gamma, beta, count)
    if last:
        return _bn_only(y2, sc2, sh2, td), None
    feat, pooled = _bn_pool(y2, sc2, sh2, td)
    return feat, pooled


def kernel(x, b0_w1, b0_b1, b0_w2, b0_b2, b0_gamma, b0_beta,
           b1_w1, b1_b1, b1_w2, b1_b2, b1_gamma, b1_beta):
    xc = jnp.transpose(x, (0, 2, 3, 4, 1))
    f0, pooled = _block(xc, b0_w1, b0_b1, b0_w2, b0_b2, b0_gamma, b0_beta,
                        last=False)
    f1, _ = _block(pooled, b1_w1, b1_b1, b1_w2, b1_b2, b1_gamma, b1_beta,
                   last=True)
    return [jnp.transpose(f0, (0, 4, 1, 2, 3)),
            jnp.transpose(f1, (0, 4, 1, 2, 3))]
